# baseline (speedup 1.0000x reference)
import functools
import math

import jax
import jax.numpy as jnp
from jax import lax
from jax.experimental import pallas as pl
from jax.experimental.pallas import tpu as pltpu
from jax.experimental.pallas import tpu_sc as plsc

B = 32
L = 1024
D = 512
TOPK = 13
KERNEL = 25
KF = 520
PACK = 1152
LANES = 16


def _movavg_sub(s):
    acc = s
    for o in range(1, KERNEL // 2 + 1):
        zo = jnp.zeros((o, D), dtype=s.dtype)
        acc = acc + jnp.concatenate([s[o:], zo], axis=0)
        acc = acc + jnp.concatenate([zo, s[:-o]], axis=0)
    return s - acc * (1.0 / KERNEL)


def _rev_rows(a):
    n = a.shape[0]
    idx = 7 - jax.lax.broadcasted_iota(jnp.int32, (8, a.shape[1]), 0)
    blocks = [
        jnp.take_along_axis(a[i * 8:(i + 1) * 8], idx, axis=0)
        for i in range(n // 8 - 1, -1, -1)
    ]
    return jnp.concatenate(blocks, axis=0)


def _corr_topk_kernel(x_ref, cf_ref, sf_ref, ct_ref, packed_ref):
    xb = x_ref[0]

    h = L // 2
    xr = _rev_rows(xb[h:])
    zpad = jnp.zeros((KF - h - 1, D), dtype=jnp.float32)
    xe = jnp.concatenate(
        [xb[:1], xb[1:h] + xr[:h - 1], xb[h:h + 1], zpad], axis=0)
    xo = jnp.concatenate(
        [jnp.zeros((1, D), jnp.float32), xb[1:h] - xr[:h - 1],
         jnp.zeros((1, D), jnp.float32), zpad], axis=0)
    re = jnp.dot(cf_ref[...], xe, preferred_element_type=jnp.float32)
    im = jnp.dot(sf_ref[...], xo, preferred_element_type=jnp.float32)
    p = re * re + im * im
    ch = jnp.dot(ct_ref[...], p, preferred_element_type=jnp.float32)
    rb = _rev_rows(ch[:h])
    corr = jnp.concatenate([ch[:h + 1], rb[:h - 1]], axis=0)

    iota_t = jax.lax.broadcasted_iota(jnp.int32, (L, D), 0)
    work = corr
    neg_inf = jnp.float32(-jnp.inf)
    vals = []
    idxs = []
    for _ in range(TOPK):
        m = jnp.max(work, axis=0, keepdims=True)
        sel = jnp.where(work == m, iota_t, L)
        ix = jnp.min(sel, axis=0, keepdims=True)
        work = jnp.where(iota_t == ix, neg_inf, work)
        vals.append(m)
        idxs.append(ix)

    vmax = vals[0]
    for v in vals[1:]:
        vmax = jnp.maximum(vmax, v)
    exps = [jnp.exp(v - vmax) for v in vals]
    denom = exps[0]
    for e in exps[1:]:
        denom = denom + e
    inv = 1.0 / denom
    weights = [e * inv for e in exps]

    zf = jnp.zeros((16 - TOPK, D), dtype=jnp.float32)
    zi = jnp.zeros((16 - TOPK, D), dtype=jnp.int32)
    w16 = jnp.concatenate(weights + [zf], axis=0)
    i16 = jnp.concatenate(idxs + [zi], axis=0)
    packed_ref[0] = jnp.concatenate(
        [
            jnp.transpose(xb),
            jnp.transpose(w16),
            jnp.transpose(i16).astype(jnp.float32),
            jnp.zeros((D, PACK - L - 32), dtype=jnp.float32),
        ],
        axis=1,
    )


def _sc_splat(vec, lane):
    dn = lax.GatherDimensionNumbers(
        offset_dims=(), collapsed_slice_dims=(0,), start_index_map=(0,))
    return lax.gather(
        vec, jnp.full((LANES, 1), lane, jnp.int32), dn, slice_sizes=(1,),
        mode=lax.GatherScatterMode.PROMISE_IN_BOUNDS)


def _make_sc_combine(rows):
    info = plsc.get_sparse_core_info()
    nw = info.num_cores * info.num_subcores
    rows_per_w = rows // nw
    mesh = plsc.VectorSubcoreMesh(core_axis_name="c", subcore_axis_name="s")

    @functools.partial(
        pl.kernel,
        mesh=mesh,
        compiler_params=pltpu.CompilerParams(needs_layout_passes=False),
        out_type=jax.ShapeDtypeStruct((rows, L), jnp.float32),
        scratch_types=[
            pltpu.VMEM((2 * L,), jnp.float32),
            pltpu.VMEM((2 * L,), jnp.float32),
            pltpu.VMEM((L,), jnp.float32),
            pltpu.VMEM((L,), jnp.float32),
            pltpu.SemaphoreType.DMA,
            pltpu.SemaphoreType.DMA,
            pltpu.SemaphoreType.DMA,
            pltpu.SemaphoreType.DMA,
        ],
    )
    def combine(packed_hbm, out_hbm, rv0, rv1, ov0, ov1, is0, is1, os0, os1):
        wid = lax.axis_index("s") * info.num_cores + lax.axis_index("c")
        base = wid * rows_per_w
        iota16 = lax.iota(jnp.int32, LANES)
        bufs = ((rv0, ov0, is0, os0), (rv1, ov1, is1, os1))
        nbuf = len(bufs)

        for b, (rvb, _, isb, _) in enumerate(bufs):
            pltpu.async_copy(
                packed_hbm.at[base + b], rvb.at[pl.ds(0, PACK)], isb)

        def row_body(j, carry):
            for b, (rvb, ovb, isb, osb) in enumerate(bufs):
                r = j * nbuf + b
                row = base + r
                pltpu.make_async_copy(
                    packed_hbm.at[row], rvb.at[pl.ds(0, PACK)], isb).wait()

                wvec = rvb[pl.ds(L, LANES)]
                dvec = rvb[pl.ds(L + LANES, LANES)].astype(jnp.int32)
                wspl = [_sc_splat(wvec, i) for i in range(TOPK)]
                dspl = [_sc_splat(dvec, i) for i in range(TOPK)]
                last = _sc_splat(rvb[pl.ds(L - LANES, LANES)], LANES - 1)
                for c in range(L // LANES):
                    rvb[pl.ds(L + c * LANES, LANES)] = last

                @pl.when(j > 0)
                def _drain():
                    pltpu.make_async_copy(ovb, out_hbm.at[row], osb).wait()

                def chunk_body(cc, carry2):
                    for u in range(8):
                        c = cc * 8 + u
                        tvec = iota16 + c * LANES
                        acc0 = jnp.zeros((LANES,), jnp.float32)
                        acc1 = jnp.zeros((LANES,), jnp.float32)
                        for i in range(TOPK):
                            g = plsc.load_gather(rvb, [tvec + dspl[i]])
                            if i % 2 == 0:
                                acc0 = acc0 + wspl[i] * g
                            else:
                                acc1 = acc1 + wspl[i] * g
                        ovb[pl.ds(c * LANES, LANES)] = acc0 + acc1
                    return carry2

                lax.fori_loop(0, L // LANES // 8, chunk_body, 0)
                pltpu.async_copy(ovb, out_hbm.at[row], osb)
                nxt = base + jnp.minimum(r + nbuf, rows_per_w - 1)
                pltpu.async_copy(
                    packed_hbm.at[nxt], rvb.at[pl.ds(0, PACK)], isb)
            return carry

        lax.fori_loop(0, rows_per_w // nbuf, row_body, 0)
        for b, (rvb, ovb, isb, osb) in enumerate(bufs):
            pltpu.make_async_copy(
                packed_hbm.at[base], rvb.at[pl.ds(0, PACK)], isb).wait()
            pltpu.make_async_copy(ovb, out_hbm.at[base], osb).wait()

    return combine


def _tail_kernel(act_ref, x_ref, w1t_ref, b1_ref, w2t_ref, b2_ref, out_ref):
    ac = jnp.transpose(act_ref[0])
    x_s = _movavg_sub(ac + x_ref[0])
    h = jnp.dot(x_s, w1t_ref[...], preferred_element_type=jnp.float32)
    h = jnp.maximum(h + b1_ref[...], 0.0)
    ff = jnp.dot(h, w2t_ref[...], preferred_element_type=jnp.float32)
    ff = ff + b2_ref[...]
    out_ref[0] = _movavg_sub(ff + x_s)


def kernel(x, W1, b1, W2, b2):
    k_idx = jnp.arange(KF, dtype=jnp.float32)[:, None]
    t_idx = jnp.arange(KF, dtype=jnp.float32)[None, :]
    ang = (2.0 * math.pi / L) * k_idx * t_idx
    valid = ((k_idx <= L // 2) & (t_idx <= L // 2)).astype(jnp.float32)
    cf = jnp.cos(ang) * valid
    sf = jnp.sin(ang) * valid
    scale = jnp.where(
        (k_idx == 0) | (k_idx == L // 2), 1.0 / L, 2.0 / L) * valid
    ct = jnp.cos(ang) * scale.T

    nb = x.shape[0]

    def corr_topk(xh):
        nh = xh.shape[0]
        return pl.pallas_call(
            _corr_topk_kernel,
            grid=(nh,),
            in_specs=[
                pl.BlockSpec((1, L, D), lambda b: (b, 0, 0)),
                pl.BlockSpec((KF, KF), lambda b: (0, 0)),
                pl.BlockSpec((KF, KF), lambda b: (0, 0)),
                pl.BlockSpec((KF, KF), lambda b: (0, 0)),
            ],
            out_specs=pl.BlockSpec((1, D, PACK), lambda b: (b, 0, 0)),
            out_shape=jax.ShapeDtypeStruct((nh, D, PACK), jnp.float32),
        )(xh, cf, sf, ct)

    def tail(acth, xh):
        nh = xh.shape[0]
        return pl.pallas_call(
            _tail_kernel,
            grid=(nh,),
            in_specs=[
                pl.BlockSpec((1, D, L), lambda b: (b, 0, 0)),
                pl.BlockSpec((1, L, D), lambda b: (b, 0, 0)),
                pl.BlockSpec((D, D), lambda b: (0, 0)),
                pl.BlockSpec((1, D), lambda b: (0, 0)),
                pl.BlockSpec((D, D), lambda b: (0, 0)),
                pl.BlockSpec((1, D), lambda b: (0, 0)),
            ],
            out_specs=pl.BlockSpec((1, L, D), lambda b: (b, 0, 0)),
            out_shape=jax.ShapeDtypeStruct((nh, L, D), jnp.float32),
        )(acth, xh, W1.T, b1[None, :], W2.T, b2[None, :])

    nsplit = 4 if nb % 4 == 0 else 1
    nh = nb // nsplit
    outs = []
    for h in range(nsplit):
        xh = x[h * nh:(h + 1) * nh]
        packed = corr_topk(xh)
        acth = _make_sc_combine(nh * D)(packed.reshape(nh * D, PACK))
        outs.append(tail(acth.reshape(nh, D, L), xh))
    return jnp.concatenate(outs, axis=0) if nsplit > 1 else outs[0]


# speedup vs baseline: 1.0256x; 1.0256x over previous
import functools
import math

import jax
import jax.numpy as jnp
from jax import lax
from jax.experimental import pallas as pl
from jax.experimental.pallas import tpu as pltpu
from jax.experimental.pallas import tpu_sc as plsc

B = 32
L = 1024
D = 512
TOPK = 13
KERNEL = 25
KF = 520
PACK = 1152
LANES = 16


def _movavg_sub(s):
    acc = s
    for o in range(1, KERNEL // 2 + 1):
        zo = jnp.zeros((o, D), dtype=s.dtype)
        acc = acc + jnp.concatenate([s[o:], zo], axis=0)
        acc = acc + jnp.concatenate([zo, s[:-o]], axis=0)
    return s - acc * (1.0 / KERNEL)


def _rev_rows(a):
    n = a.shape[0]
    idx = 7 - jax.lax.broadcasted_iota(jnp.int32, (8, a.shape[1]), 0)
    blocks = [
        jnp.take_along_axis(a[i * 8:(i + 1) * 8], idx, axis=0)
        for i in range(n // 8 - 1, -1, -1)
    ]
    return jnp.concatenate(blocks, axis=0)


def _corr_topk_kernel(x_ref, cf_ref, sf_ref, ct_ref, packed_ref):
    xb = x_ref[0]

    h = L // 2
    xr = _rev_rows(xb[h:])
    zpad = jnp.zeros((KF - h - 1, D), dtype=jnp.float32)
    xe = jnp.concatenate(
        [xb[:1], xb[1:h] + xr[:h - 1], xb[h:h + 1], zpad], axis=0)
    xo = jnp.concatenate(
        [jnp.zeros((1, D), jnp.float32), xb[1:h] - xr[:h - 1],
         jnp.zeros((1, D), jnp.float32), zpad], axis=0)
    re = jnp.dot(cf_ref[...], xe, preferred_element_type=jnp.float32)
    im = jnp.dot(sf_ref[...], xo, preferred_element_type=jnp.float32)
    p = re * re + im * im
    ch = jnp.dot(ct_ref[...], p, preferred_element_type=jnp.float32)
    rb = _rev_rows(ch[:h])
    corr = jnp.concatenate([ch[:h + 1], rb[:h - 1]], axis=0)

    iota_t = jax.lax.broadcasted_iota(jnp.int32, (L, D), 0)
    work = corr
    neg_inf = jnp.float32(-jnp.inf)
    vals = []
    idxs = []
    for _ in range(TOPK):
        m = jnp.max(work, axis=0, keepdims=True)
        sel = jnp.where(work == m, iota_t, L)
        ix = jnp.min(sel, axis=0, keepdims=True)
        work = jnp.where(iota_t == ix, neg_inf, work)
        vals.append(m)
        idxs.append(ix)

    vmax = vals[0]
    for v in vals[1:]:
        vmax = jnp.maximum(vmax, v)
    exps = [jnp.exp(v - vmax) for v in vals]
    denom = exps[0]
    for e in exps[1:]:
        denom = denom + e
    inv = 1.0 / denom
    weights = [e * inv for e in exps]

    zf = jnp.zeros((16 - TOPK, D), dtype=jnp.float32)
    zi = jnp.zeros((16 - TOPK, D), dtype=jnp.int32)
    w16 = jnp.concatenate(weights + [zf], axis=0)
    i16 = jnp.concatenate(idxs + [zi], axis=0)
    packed_ref[0] = jnp.concatenate(
        [
            jnp.transpose(xb),
            jnp.transpose(w16),
            jnp.transpose(i16).astype(jnp.float32),
            jnp.zeros((D, PACK - L - 32), dtype=jnp.float32),
        ],
        axis=1,
    )


def _sc_splat(vec, lane):
    dn = lax.GatherDimensionNumbers(
        offset_dims=(), collapsed_slice_dims=(0,), start_index_map=(0,))
    return lax.gather(
        vec, jnp.full((LANES, 1), lane, jnp.int32), dn, slice_sizes=(1,),
        mode=lax.GatherScatterMode.PROMISE_IN_BOUNDS)


def _make_sc_combine(rows):
    info = plsc.get_sparse_core_info()
    nw = info.num_cores * info.num_subcores
    rows_per_w = rows // nw
    mesh = plsc.VectorSubcoreMesh(core_axis_name="c", subcore_axis_name="s")

    @functools.partial(
        pl.kernel,
        mesh=mesh,
        compiler_params=pltpu.CompilerParams(needs_layout_passes=False),
        out_type=jax.ShapeDtypeStruct((rows, L), jnp.float32),
        scratch_types=[
            pltpu.VMEM((2 * L,), jnp.float32),
            pltpu.VMEM((2 * L,), jnp.float32),
            pltpu.VMEM((L,), jnp.float32),
            pltpu.VMEM((L,), jnp.float32),
            pltpu.SemaphoreType.DMA,
            pltpu.SemaphoreType.DMA,
            pltpu.SemaphoreType.DMA,
            pltpu.SemaphoreType.DMA,
        ],
    )
    def combine(packed_hbm, out_hbm, rv0, rv1, ov0, ov1, is0, is1, os0, os1):
        wid = lax.axis_index("s") * info.num_cores + lax.axis_index("c")
        base = wid * rows_per_w
        iota16 = lax.iota(jnp.int32, LANES)
        bufs = ((rv0, ov0, is0, os0), (rv1, ov1, is1, os1))
        nbuf = len(bufs)

        for b, (rvb, _, isb, _) in enumerate(bufs):
            pltpu.async_copy(
                packed_hbm.at[base + b], rvb.at[pl.ds(0, PACK)], isb)

        def row_body(j, carry):
            for b, (rvb, ovb, isb, osb) in enumerate(bufs):
                r = j * nbuf + b
                row = base + r
                pltpu.make_async_copy(
                    packed_hbm.at[row], rvb.at[pl.ds(0, PACK)], isb).wait()

                wvec = rvb[pl.ds(L, LANES)]
                dvec = rvb[pl.ds(L + LANES, LANES)].astype(jnp.int32)
                wspl = [_sc_splat(wvec, i) for i in range(TOPK)]
                dspl = [_sc_splat(dvec, i) for i in range(TOPK)]
                last = _sc_splat(rvb[pl.ds(L - LANES, LANES)], LANES - 1)
                for c in range(L // LANES):
                    rvb[pl.ds(L + c * LANES, LANES)] = last

                @pl.when(j > 0)
                def _drain():
                    pltpu.make_async_copy(ovb, out_hbm.at[row], osb).wait()

                def chunk_body(cc, carry2):
                    for u in range(8):
                        c = cc * 8 + u
                        tvec = iota16 + c * LANES
                        acc0 = jnp.zeros((LANES,), jnp.float32)
                        acc1 = jnp.zeros((LANES,), jnp.float32)
                        for i in range(TOPK):
                            g = plsc.load_gather(rvb, [tvec + dspl[i]])
                            if i % 2 == 0:
                                acc0 = acc0 + wspl[i] * g
                            else:
                                acc1 = acc1 + wspl[i] * g
                        ovb[pl.ds(c * LANES, LANES)] = acc0 + acc1
                    return carry2

                lax.fori_loop(0, L // LANES // 8, chunk_body, 0)
                pltpu.async_copy(ovb, out_hbm.at[row], osb)
                nxt = base + jnp.minimum(r + nbuf, rows_per_w - 1)
                pltpu.async_copy(
                    packed_hbm.at[nxt], rvb.at[pl.ds(0, PACK)], isb)
            return carry

        lax.fori_loop(0, rows_per_w // nbuf, row_body, 0)
        for b, (rvb, ovb, isb, osb) in enumerate(bufs):
            pltpu.make_async_copy(
                packed_hbm.at[base], rvb.at[pl.ds(0, PACK)], isb).wait()
            pltpu.make_async_copy(ovb, out_hbm.at[base], osb).wait()

    return combine


def _tail_kernel(act_ref, x_ref, w1t_ref, b1_ref, w2t_ref, b2_ref, out_ref):
    ac = jnp.transpose(act_ref[0])
    x_s = _movavg_sub(ac + x_ref[0])
    h = jnp.dot(x_s, w1t_ref[...], preferred_element_type=jnp.float32)
    h = jnp.maximum(h + b1_ref[...], 0.0)
    ff = jnp.dot(h, w2t_ref[...], preferred_element_type=jnp.float32)
    ff = ff + b2_ref[...]
    out_ref[0] = _movavg_sub(ff + x_s)


def kernel(x, W1, b1, W2, b2):
    k_idx = jnp.arange(KF, dtype=jnp.float32)[:, None]
    t_idx = jnp.arange(KF, dtype=jnp.float32)[None, :]
    ang = (2.0 * math.pi / L) * k_idx * t_idx
    valid = ((k_idx <= L // 2) & (t_idx <= L // 2)).astype(jnp.float32)
    cf = jnp.cos(ang) * valid
    sf = jnp.sin(ang) * valid
    scale = jnp.where(
        (k_idx == 0) | (k_idx == L // 2), 1.0 / L, 2.0 / L) * valid
    ct = jnp.cos(ang) * scale.T

    nb = x.shape[0]

    def corr_topk(xh):
        nh = xh.shape[0]
        return pl.pallas_call(
            _corr_topk_kernel,
            grid=(nh,),
            in_specs=[
                pl.BlockSpec((1, L, D), lambda b: (b, 0, 0)),
                pl.BlockSpec((KF, KF), lambda b: (0, 0)),
                pl.BlockSpec((KF, KF), lambda b: (0, 0)),
                pl.BlockSpec((KF, KF), lambda b: (0, 0)),
            ],
            out_specs=pl.BlockSpec((1, D, PACK), lambda b: (b, 0, 0)),
            out_shape=jax.ShapeDtypeStruct((nh, D, PACK), jnp.float32),
        )(xh, cf, sf, ct)

    def tail(acth, xh):
        nh = xh.shape[0]
        return pl.pallas_call(
            _tail_kernel,
            grid=(nh,),
            in_specs=[
                pl.BlockSpec((1, D, L), lambda b: (b, 0, 0)),
                pl.BlockSpec((1, L, D), lambda b: (b, 0, 0)),
                pl.BlockSpec((D, D), lambda b: (0, 0)),
                pl.BlockSpec((1, D), lambda b: (0, 0)),
                pl.BlockSpec((D, D), lambda b: (0, 0)),
                pl.BlockSpec((1, D), lambda b: (0, 0)),
            ],
            out_specs=pl.BlockSpec((1, L, D), lambda b: (b, 0, 0)),
            out_shape=jax.ShapeDtypeStruct((nh, L, D), jnp.float32),
        )(acth, xh, W1.T, b1[None, :], W2.T, b2[None, :])

    nsplit = 8 if nb % 8 == 0 else 1
    nh = nb // nsplit
    outs = []
    for h in range(nsplit):
        xh = x[h * nh:(h + 1) * nh]
        packed = corr_topk(xh)
        acth = _make_sc_combine(nh * D)(packed.reshape(nh * D, PACK))
        outs.append(tail(acth.reshape(nh, D, L), xh))
    return jnp.concatenate(outs, axis=0) if nsplit > 1 else outs[0]
